# baseline (speedup 1.0000x reference)
import jax
import jax.numpy as jnp
from jax.experimental import pallas as pl
from jax.experimental.pallas import tpu as pltpu

EPS = 1e-7


def _matvec_body(x_ref, w_ref, o_ref):
    o_ref[...] = jnp.dot(x_ref[...], w_ref[...],
                         preferred_element_type=jnp.float32)


def _loss_body(nblocks, bj, t_row_ref, lh_row_ref, t_col_ref, lh_col_ref,
               e_col_ref, o_ref, exp_s, acc_s, sc_s):
    s = pl.program_id(0)
    b = t_row_ref.shape[1]
    nj = b // bj

    @pl.when(s == 0)
    def _init():
        lr = lh_row_ref[...]
        gamma = jnp.max(lr)
        sc_s[0] = gamma
        exp_s[...] = jnp.exp(lr - gamma)

    gamma = sc_s[0]
    ti = t_col_ref[...]
    for c in range(nj):
        tj = t_row_ref[0, c * bj:(c + 1) * bj]
        ej = exp_s[0, c * bj:(c + 1) * bj]
        v = jnp.where(tj[None, :] >= ti, ej[None, :], 0.0)
        if c == 0:
            acc_s[...] = v
        else:
            acc_s[...] = acc_s[...] + v
    denom = jnp.sum(acc_s[...], axis=1, keepdims=True)
    log_cumsum = jnp.log(denom + EPS) + gamma
    e_blk = e_col_ref[...]
    part = jnp.sum(e_blk * (lh_col_ref[...] - log_cumsum))
    esum = jnp.sum(e_blk)
    sc_s[1] = jnp.where(s == 0, part, sc_s[1] + part)
    sc_s[2] = jnp.where(s == 0, esum, sc_s[2] + esum)
    o_ref[...] = jnp.broadcast_to(-(sc_s[1] / sc_s[2]), (1, 1))


def kernel(x, y_true, W):
    b, d = x.shape
    bi1 = 2048
    bi = 1024
    bj = 2048

    logh = pl.pallas_call(
        _matvec_body,
        grid=(b // bi1,),
        in_specs=[
            pl.BlockSpec((bi1, d), lambda s: (s, 0)),
            pl.BlockSpec((d, 1), lambda s: (0, 0)),
        ],
        out_specs=pl.BlockSpec((bi1, 1), lambda s: (s, 0)),
        out_shape=jax.ShapeDtypeStruct((b, 1), jnp.float32),
    )(x, W)

    t = y_true[:, 0]
    e = y_true[:, 1]
    t_row = t.reshape(1, b)
    lh_row = logh.reshape(1, b)
    t_col = t.reshape(b, 1)
    e_col = e.reshape(b, 1)

    import functools
    body = functools.partial(_loss_body, b // bi, bj)
    out = pl.pallas_call(
        body,
        grid=(b // bi,),
        in_specs=[
            pl.BlockSpec((1, b), lambda s: (0, 0)),
            pl.BlockSpec((1, b), lambda s: (0, 0)),
            pl.BlockSpec((bi, 1), lambda s: (s, 0)),
            pl.BlockSpec((bi, 1), lambda s: (s, 0)),
            pl.BlockSpec((bi, 1), lambda s: (s, 0)),
        ],
        out_specs=pl.BlockSpec((1, 1), lambda s: (0, 0)),
        out_shape=jax.ShapeDtypeStruct((1, 1), jnp.float32),
        scratch_shapes=[
            pltpu.VMEM((1, b), jnp.float32),
            pltpu.VMEM((bi, bj), jnp.float32),
            pltpu.SMEM((3,), jnp.float32),
        ],
    )(t_row, lh_row, t_col, logh, e_col)

    return out[0, 0]


# speedup vs baseline: 4.7380x; 4.7380x over previous
import jax
import jax.numpy as jnp
from jax.experimental import pallas as pl
from jax.experimental.pallas import tpu as pltpu

EPS = 1e-7
R = 128


def _matvec_body(x_ref, w_ref, o_ref):
    o_ref[...] = jnp.dot(x_ref[...], w_ref[...],
                         preferred_element_type=jnp.float32)


def _partner(x, i_low, sh, ax):
    return jnp.where(i_low, pltpu.roll(x, R - sh, ax), pltpu.roll(x, sh, ax))


def _loss_body(t_ref, lh_ref, e_ref, o_ref):
    t = t_ref[...]
    lh = lh_ref[...]
    e = e_ref[...]
    gamma = jnp.max(lh)
    ex = jnp.exp(lh - gamma)
    s_elh = jnp.sum(e * lh)
    s_e = jnp.sum(e)

    lin = (jax.lax.broadcasted_iota(jnp.int32, (R, R), 0) * R
           + jax.lax.broadcasted_iota(jnp.int32, (R, R), 1))

    n = R * R
    k = 2
    while k <= n:
        j = k // 2
        while j >= 1:
            up = (lin & k) != 0
            i_low = (lin & j) == 0
            if j >= R:
                ax, sh = 0, j // R
            else:
                ax, sh = 1, j
            tp = _partner(t, i_low, sh, ax)
            take_min = i_low == up
            sel_p = (take_min & (tp < t)) | (~take_min & (tp > t))
            t = jnp.where(sel_p, tp, t)
            ex = jnp.where(sel_p, _partner(ex, i_low, sh, ax), ex)
            e = jnp.where(sel_p, _partner(e, i_low, sh, ax), e)
            j //= 2
        k *= 2

    ri = jax.lax.broadcasted_iota(jnp.int32, (R, R), 0)
    ci = jax.lax.broadcasted_iota(jnp.int32, (R, R), 1)
    tri_incl = (ri <= ci).astype(jnp.float32)
    tri_strict = (ci < ri).astype(jnp.float32)
    csum = jnp.dot(ex, tri_incl, preferred_element_type=jnp.float32)
    rowsum = csum[:, R - 1:R]
    off = jnp.dot(tri_strict, rowsum, preferred_element_type=jnp.float32)
    denom = csum + off
    lcs = jnp.log(denom + EPS) + gamma
    num = s_elh - jnp.sum(e * lcs)
    o_ref[...] = jnp.broadcast_to(-(num / s_e), (1, 1))


def kernel(x, y_true, W):
    b, d = x.shape
    bi1 = 2048

    logh = pl.pallas_call(
        _matvec_body,
        grid=(b // bi1,),
        in_specs=[
            pl.BlockSpec((bi1, d), lambda s: (s, 0)),
            pl.BlockSpec((d, 1), lambda s: (0, 0)),
        ],
        out_specs=pl.BlockSpec((bi1, 1), lambda s: (s, 0)),
        out_shape=jax.ShapeDtypeStruct((b, 1), jnp.float32),
    )(x, W)

    t2 = y_true[:, 0].reshape(R, R)
    e2 = y_true[:, 1].reshape(R, R)
    lh2 = logh.reshape(R, R)

    out = pl.pallas_call(
        _loss_body,
        out_shape=jax.ShapeDtypeStruct((1, 1), jnp.float32),
    )(t2, lh2, e2)

    return out[0, 0]


# speedup vs baseline: 6.3642x; 1.3432x over previous
import jax
import jax.numpy as jnp
from jax.experimental import pallas as pl
from jax.experimental.pallas import tpu as pltpu

EPS = 1e-7
NR = 8
NC = 2048
N = NR * NC


def _partner(x, i_low, sh, ax, size):
    return jnp.where(i_low, pltpu.roll(x, size - sh, ax), pltpu.roll(x, sh, ax))


def _fused_body(x_ref, w_ref, t_ref, e_ref, o_ref, lh_s):
    s = pl.program_id(0)

    @pl.when(s < NR)
    def _matvec():
        row = jax.lax.dot_general(
            w_ref[...], x_ref[...], (((1,), (1,)), ((), ())),
            preferred_element_type=jnp.float32)
        lh_s[pl.ds(s, 1), :] = row

    @pl.when(s == NR)
    def _loss():
        t = t_ref[...]
        e = e_ref[...]
        lh = lh_s[...]
        gamma = jnp.max(lh)
        ex = jnp.exp(lh - gamma)
        s_elh = jnp.sum(e * lh)
        s_e = jnp.sum(e)

        exb = jax.lax.bitcast_convert_type(
            ex.astype(jnp.bfloat16), jnp.uint16).astype(jnp.uint32)
        eb = jax.lax.bitcast_convert_type(
            e.astype(jnp.bfloat16), jnp.uint16).astype(jnp.uint32)
        pay = (exb << 16) | eb

        ri = jax.lax.broadcasted_iota(jnp.int32, (NR, NC), 0)
        ci = jax.lax.broadcasted_iota(jnp.int32, (NR, NC), 1)
        lin = ri * NC + ci

        tk = t
        k = 2
        while k <= N:
            j = k // 2
            while j >= 1:
                up = (lin & k) != 0
                i_low = (lin & j) == 0
                if j >= NC:
                    ax, sh, size = 0, j // NC, NR
                else:
                    ax, sh, size = 1, j, NC
                tp = _partner(tk, i_low, sh, ax, size)
                take_min = i_low == up
                sel_p = (take_min & (tp < tk)) | (~take_min & (tp > tk))
                tk = jnp.where(sel_p, tp, tk)
                pay = jnp.where(sel_p, _partner(pay, i_low, sh, ax, size), pay)
                j //= 2
            k *= 2

        exs = jax.lax.bitcast_convert_type(
            (pay >> 16).astype(jnp.uint16), jnp.bfloat16).astype(jnp.float32)
        es = jax.lax.bitcast_convert_type(
            (pay & 0xFFFF).astype(jnp.uint16), jnp.bfloat16).astype(jnp.float32)

        csum = exs
        d = 1
        while d < NC:
            csum = csum + jnp.where(ci >= d, pltpu.roll(csum, d, 1), 0.0)
            d *= 2
        rowtot = csum[:, NC - 1:NC]
        ri8 = jax.lax.broadcasted_iota(jnp.int32, (NR, 1), 0)
        off = jnp.where(ri8 >= 1, pltpu.roll(rowtot, 1, 0), 0.0)
        d = 1
        while d < NR:
            off = off + jnp.where(ri8 >= d, pltpu.roll(off, d, 0), 0.0)
            d *= 2
        denom = csum + off
        lcs = jnp.log(denom + EPS) + gamma
        num = s_elh - jnp.sum(es * lcs)
        o_ref[...] = jnp.broadcast_to(-(num / s_e), (1, 1))


def kernel(x, y_true, W):
    b, d = x.shape
    bi1 = b // NR

    t2 = y_true[:, 0].reshape(NR, NC)
    e2 = y_true[:, 1].reshape(NR, NC)
    w_row = W.reshape(1, d)

    out = pl.pallas_call(
        _fused_body,
        grid=(NR + 1,),
        in_specs=[
            pl.BlockSpec((bi1, d), lambda s: (jnp.minimum(s, NR - 1), 0)),
            pl.BlockSpec((1, d), lambda s: (0, 0)),
            pl.BlockSpec((NR, NC), lambda s: (0, 0)),
            pl.BlockSpec((NR, NC), lambda s: (0, 0)),
        ],
        out_specs=pl.BlockSpec((1, 1), lambda s: (0, 0)),
        out_shape=jax.ShapeDtypeStruct((1, 1), jnp.float32),
        scratch_shapes=[pltpu.VMEM((NR, NC), jnp.float32)],
    )(x, w_row, t2, e2)

    return out[0, 0]


# speedup vs baseline: 6.6660x; 1.0474x over previous
import jax
import jax.numpy as jnp
from jax.experimental import pallas as pl
from jax.experimental.pallas import tpu as pltpu

EPS = 1e-7
NR = 8
NC = 2048
N = NR * NC


def _partner(x, i_low, sh, ax, size):
    return jnp.where(i_low, pltpu.roll(x, size - sh, ax), pltpu.roll(x, sh, ax))


def _fused_body(x_ref, w_ref, t_ref, e_ref, o_ref, lh_s):
    s = pl.program_id(0)

    @pl.when(s < NR)
    def _matvec():
        row = jax.lax.dot_general(
            w_ref[...], x_ref[...], (((1,), (1,)), ((), ())),
            preferred_element_type=jnp.float32)
        lh_s[pl.ds(s, 1), :] = row

    @pl.when(s == NR)
    def _loss():
        t = t_ref[...]
        e = e_ref[...]
        lh = lh_s[...]
        gamma = jnp.max(lh)
        ex = jnp.exp(lh - gamma)
        s_elh = jnp.sum(e * lh)
        s_e = jnp.sum(e)

        exb = jax.lax.bitcast_convert_type(
            ex.astype(jnp.bfloat16), jnp.uint16).astype(jnp.uint32)
        eb = jax.lax.bitcast_convert_type(
            e.astype(jnp.bfloat16), jnp.uint16).astype(jnp.uint32)
        pay = (exb << 16) | eb

        ri = jax.lax.broadcasted_iota(jnp.int32, (NR, NC), 0)
        ci = jax.lax.broadcasted_iota(jnp.int32, (NR, NC), 1)
        lin = ri * NC + ci

        tk = t
        k = 2
        while k <= N:
            j = k // 2
            while j >= 1:
                up = (lin & k) != 0
                i_low = (lin & j) == 0
                if j >= NC:
                    ax, sh, size = 0, j // NC, NR
                else:
                    ax, sh, size = 1, j, NC
                tp = _partner(tk, i_low, sh, ax, size)
                take_min = i_low == up
                sel_p = (tp < tk) == take_min
                tk = jnp.where(sel_p, tp, tk)
                pay = jnp.where(sel_p, _partner(pay, i_low, sh, ax, size), pay)
                j //= 2
            k *= 2

        exs = jax.lax.bitcast_convert_type(
            (pay >> 16).astype(jnp.uint16), jnp.bfloat16).astype(jnp.float32)
        es = jax.lax.bitcast_convert_type(
            (pay & 0xFFFF).astype(jnp.uint16), jnp.bfloat16).astype(jnp.float32)

        csum = exs
        d = 1
        while d < NC:
            csum = csum + jnp.where(ci >= d, pltpu.roll(csum, d, 1), 0.0)
            d *= 2
        rowtot = csum[:, NC - 1:NC]
        ri8 = jax.lax.broadcasted_iota(jnp.int32, (NR, 1), 0)
        off = jnp.where(ri8 >= 1, pltpu.roll(rowtot, 1, 0), 0.0)
        d = 1
        while d < NR:
            off = off + jnp.where(ri8 >= d, pltpu.roll(off, d, 0), 0.0)
            d *= 2
        denom = csum + off
        lcs = jnp.log(denom + EPS) + gamma
        num = s_elh - jnp.sum(es * lcs)
        o_ref[...] = jnp.broadcast_to(-(num / s_e), (1, 1))


def kernel(x, y_true, W):
    b, d = x.shape
    bi1 = b // NR

    t2 = y_true[:, 0].reshape(NR, NC)
    e2 = y_true[:, 1].reshape(NR, NC)
    w_row = W.reshape(1, d)

    out = pl.pallas_call(
        _fused_body,
        grid=(NR + 1,),
        in_specs=[
            pl.BlockSpec((bi1, d), lambda s: (jnp.minimum(s, NR - 1), 0)),
            pl.BlockSpec((1, d), lambda s: (0, 0)),
            pl.BlockSpec((NR, NC), lambda s: (0, 0)),
            pl.BlockSpec((NR, NC), lambda s: (0, 0)),
        ],
        out_specs=pl.BlockSpec((1, 1), lambda s: (0, 0)),
        out_shape=jax.ShapeDtypeStruct((1, 1), jnp.float32),
        scratch_shapes=[pltpu.VMEM((NR, NC), jnp.float32)],
    )(x, w_row, t2, e2)

    return out[0, 0]
